# baseline (speedup 1.0000x reference)
import functools

import jax
import jax.numpy as jnp
from jax import lax
from jax.experimental import pallas as pl
from jax.experimental.pallas import tpu as pltpu
from jax.experimental.pallas import tpu_sc as plsc

B = 4096
S = 200
D = 64
NC = 2
NS = 16
NW = NC * NS
ROWS = B * S
RPW = ROWS // NW
C = S
NCH = RPW // C
SCALE = 8.0


def _mesh():
    return plsc.VectorSubcoreMesh(core_axis_name="c", subcore_axis_name="s")


@functools.partial(
    pl.kernel,
    mesh=_mesh(),
    out_type=jax.ShapeDtypeStruct((ROWS, D), jnp.float32),
    compiler_params=pltpu.CompilerParams(use_tc_tiling_on_sc=False),
    scratch_types=[
        pltpu.VMEM((C,), jnp.int32),
        pltpu.VMEM((C, D), jnp.float32),
        pltpu.VMEM((S, D), jnp.float32),
        pltpu.SemaphoreType.DMA,
    ],
)
def _embed(idx_hbm, tok_hbm, pos_hbm, out_hbm, idx_v, rows_v, pos_v, sem):
    wid = lax.axis_index("s") * NC + lax.axis_index("c")
    base = wid * RPW
    pltpu.sync_copy(pos_hbm, pos_v)

    def chunk_body(c, carry):
        row0 = base + c * C
        pltpu.sync_copy(idx_hbm.at[pl.ds(row0, C)], idx_v)
        pltpu.async_copy(tok_hbm.at[idx_v], rows_v, sem).wait()

        def row_body(r, carry2):
            for j in range(D // 16):
                sl = pl.ds(j * 16, 16)
                rows_v[r, sl] = rows_v[r, sl] * SCALE + pos_v[r, sl]
            return carry2

        lax.fori_loop(0, C, row_body, 0, unroll=2)
        pltpu.sync_copy(rows_v, out_hbm.at[pl.ds(row0, C)])
        return carry

    lax.fori_loop(0, NCH, chunk_body, 0)


def kernel(inputs, tok_table, pos_table):
    idx = inputs.reshape(ROWS).astype(jnp.int32)
    out = _embed(idx, tok_table, pos_table)
    return out.reshape(B, S, D)


# speedup vs baseline: 1.5596x; 1.5596x over previous
import functools

import jax
import jax.numpy as jnp
from jax import lax
from jax.experimental import pallas as pl
from jax.experimental.pallas import tpu as pltpu
from jax.experimental.pallas import tpu_sc as plsc

B = 4096
S = 200
D = 64
NC = 2
NS = 16
NW = NC * NS
ROWS = B * S
RPW = ROWS // NW
C = 2 * S
NCH = RPW // C
SCALE = 8.0


def _mesh():
    return plsc.VectorSubcoreMesh(core_axis_name="c", subcore_axis_name="s")


@functools.partial(
    pl.kernel,
    mesh=_mesh(),
    out_type=jax.ShapeDtypeStruct((ROWS, D), jnp.float32),
    compiler_params=pltpu.CompilerParams(use_tc_tiling_on_sc=False),
    scratch_types=[
        pltpu.VMEM((2, C), jnp.int32),
        pltpu.VMEM((2, C, D), jnp.float32),
        pltpu.VMEM((2, C, D), jnp.float32),
        pltpu.VMEM((S, D), jnp.float32),
        pltpu.SemaphoreType.DMA,
        pltpu.SemaphoreType.DMA,
        pltpu.SemaphoreType.DMA,
        pltpu.SemaphoreType.DMA,
        pltpu.SemaphoreType.DMA,
        pltpu.SemaphoreType.DMA,
    ],
)
def _embed(idx_hbm, tok_hbm, pos_hbm, out_hbm, idx_v, g_v, o_v, pos_v,
           isem0, isem1, gsem0, gsem1, osem0, osem1):
    isem = (isem0, isem1)
    gsem = (gsem0, gsem1)
    osem = (osem0, osem1)
    wid = lax.axis_index("s") * NC + lax.axis_index("c")
    base = wid * RPW
    pltpu.sync_copy(pos_hbm, pos_v)

    for b in range(2):
        pltpu.sync_copy(idx_hbm.at[pl.ds(base + b * C, C)], idx_v.at[b])
        pltpu.make_async_copy(tok_hbm.at[idx_v.at[b]], g_v.at[b],
                              gsem[b]).start()

    def outer(gi, carry):
        for b in range(2):
            c = 2 * gi + b
            row0 = base + c * C
            pltpu.make_async_copy(tok_hbm.at[idx_v.at[b]], g_v.at[b],
                                  gsem[b]).wait()
            @pl.when(c < NCH - 2)
            def _stage():
                pltpu.make_async_copy(
                    idx_hbm.at[pl.ds(row0 + 2 * C, C)], idx_v.at[b],
                    isem[b]).start()

            @pl.when(c >= 2)
            def _drain():
                pltpu.make_async_copy(o_v.at[b], out_hbm.at[pl.ds(row0, C)],
                                      osem[b]).wait()

            def row_body(r, carry2):
                for s_blk in range(C // S):
                    row = s_blk * S + r
                    for j in range(D // 16):
                        sl = pl.ds(j * 16, 16)
                        o_v[b, row, sl] = g_v[b, row, sl] * SCALE + pos_v[r, sl]
                return carry2

            lax.fori_loop(0, S, row_body, 0, unroll=2)

            pltpu.make_async_copy(o_v.at[b], out_hbm.at[pl.ds(row0, C)],
                                  osem[b]).start()

            @pl.when(c < NCH - 2)
            def _next_gather():
                pltpu.make_async_copy(
                    idx_hbm.at[pl.ds(row0 + 2 * C, C)], idx_v.at[b],
                    isem[b]).wait()
                pltpu.make_async_copy(tok_hbm.at[idx_v.at[b]], g_v.at[b],
                                      gsem[b]).start()
        return carry

    lax.fori_loop(0, NCH // 2, outer, 0)
    for b in range(2):
        pltpu.make_async_copy(o_v.at[b], out_hbm.at[pl.ds(base, C)],
                              osem[b]).wait()


def kernel(inputs, tok_table, pos_table):
    idx = inputs.reshape(ROWS).astype(jnp.int32)
    out = _embed(idx, tok_table, pos_table)
    return out.reshape(B, S, D)
